# baseline (speedup 1.0000x reference)
import functools
import math

import jax
import jax.numpy as jnp
from jax.experimental import pallas as pl
from jax.experimental.pallas import tpu as pltpu

_H = 16
_BLK = 64
_TOPK_FRAC = 0.5


def _dot(a, b, dims):
    return jax.lax.dot_general(a, b, dims, preferred_element_type=jnp.float32)


def _qkv_kernel(x_ref, w_ref, o_ref, m_ref, *, hd, heads_per_step, blk, nb):
    r = _dot(x_ref[...], w_ref[...], (((1,), (0,)), ((), ())))
    n = r.shape[0]
    m = jnp.sum(r.reshape(nb, blk, heads_per_step * hd), axis=1) * (1.0 / blk)
    for t in range(heads_per_step):
        o_ref[t] = r[:, t * hd:(t + 1) * hd].astype(jnp.bfloat16)
        m_ref[t] = m[:, t * hd:(t + 1) * hd]


def _select_kernel(qm_ref, km_ref, idx_ref, *, nb, kc):
    qb = qm_ref[0]
    kb = km_ref[0]
    sim = _dot(qb.astype(jnp.bfloat16), kb.astype(jnp.bfloat16),
               (((1,), (1,)), ((), ())))
    j_iota = jax.lax.broadcasted_iota(jnp.int32, (nb, nb), 1)
    count = jnp.zeros((nb, nb), jnp.float32)
    for jp in range(nb):
        coljp = sim[:, jp:jp + 1]
        beats = (coljp > sim) | ((coljp == sim) & (jp < j_iota))
        count = count + beats.astype(jnp.float32)
    bmask = (count < kc).astype(jnp.float32)
    lt = (jax.lax.broadcasted_iota(jnp.int32, (nb, nb), 0)
          < jax.lax.broadcasted_iota(jnp.int32, (nb, nb), 1)).astype(jnp.float32)
    pos = jnp.dot(bmask, lt, preferred_element_type=jnp.float32,
                  precision=jax.lax.Precision.HIGHEST)
    p_iota = jax.lax.broadcasted_iota(jnp.int32, (nb, kc, nb), 1).astype(jnp.float32)
    j3 = jax.lax.broadcasted_iota(jnp.int32, (nb, kc, nb), 2).astype(jnp.float32)
    oh = ((pos[:, None, :] == p_iota) & (bmask[:, None, :] > 0)).astype(jnp.float32)
    idxf = jnp.sum(j3 * oh, axis=2)
    idx_ref[0] = idxf.astype(jnp.int32)


def _attn_kernel(idx_ref, q_ref, k_ref, v_ref, o_ref, *, scale, blk, kc, qpg):
    h = pl.program_id(0)
    qg = pl.program_id(1)
    for t in range(qpg):
        qi = qg * qpg + t
        q = q_ref[0, pl.ds(t * blk, blk), :]
        s = jnp.concatenate(
            [_dot(q, k_ref[0, pl.ds(idx_ref[h, qi, j] * blk, blk), :],
                  (((1,), (1,)), ((), ()))) for j in range(kc)],
            axis=1) * scale
        mx = jnp.max(s, axis=-1, keepdims=True)
        pexp = jnp.exp(s - mx)
        l = jnp.sum(pexp, axis=-1, keepdims=True)
        po = pexp.astype(jnp.bfloat16)
        acc = _dot(po[:, 0:blk], v_ref[0, pl.ds(idx_ref[h, qi, 0] * blk, blk), :],
                   (((1,), (0,)), ((), ())))
        for j in range(1, kc):
            acc = acc + _dot(po[:, j * blk:(j + 1) * blk],
                             v_ref[0, pl.ds(idx_ref[h, qi, j] * blk, blk), :],
                             (((1,), (0,)), ((), ())))
        o_ref[pl.ds(t * blk, blk), :] = (acc / l).astype(jnp.bfloat16)


def _proj_kernel(a_ref, w_ref, b_ref, o_ref):
    o_ref[...] = _dot(a_ref[...], w_ref[...], (((1,), (0,)), ((), ()))) + b_ref[...]


def kernel(x, W_qkv, W_proj, b_proj):
    B, N, C = x.shape
    H = _H
    hd = C // H
    blk = _BLK
    nb = N // blk
    kc = max(1, int(math.ceil(_TOPK_FRAC * nb)))
    scale = hd ** -0.5
    xb = x.reshape(N, C).astype(jnp.bfloat16)
    wqb = W_qkv.astype(jnp.bfloat16)
    wpb = W_proj.astype(jnp.bfloat16)

    hps = 4
    qkvs, means = pl.pallas_call(
        functools.partial(_qkv_kernel, hd=hd, heads_per_step=hps, blk=blk, nb=nb),
        grid=(3 * H // hps,),
        in_specs=[
            pl.BlockSpec((N, C), lambda j: (0, 0)),
            pl.BlockSpec((C, hps * hd), lambda j: (0, j)),
        ],
        out_specs=[
            pl.BlockSpec((hps, N, hd), lambda j: (j, 0, 0)),
            pl.BlockSpec((hps, nb, hd), lambda j: (j, 0, 0)),
        ],
        out_shape=[
            jax.ShapeDtypeStruct((3 * H, N, hd), jnp.bfloat16),
            jax.ShapeDtypeStruct((3 * H, nb, hd), jnp.float32),
        ],
    )(xb, wqb)

    idx = pl.pallas_call(
        functools.partial(_select_kernel, nb=nb, kc=kc),
        grid=(H,),
        in_specs=[
            pl.BlockSpec((1, nb, hd), lambda h: (h, 0, 0)),
            pl.BlockSpec((1, nb, hd), lambda h: (H + h, 0, 0)),
        ],
        out_specs=pl.BlockSpec((1, nb, kc), lambda h: (h, 0, 0)),
        out_shape=jax.ShapeDtypeStruct((H, nb, kc), jnp.int32),
    )(means, means)

    qpg = 8
    attn = pl.pallas_call(
        functools.partial(_attn_kernel, scale=scale, blk=blk, kc=kc, qpg=qpg),
        grid=(H, nb // qpg),
        in_specs=[
            pl.BlockSpec(memory_space=pltpu.SMEM),
            pl.BlockSpec((1, qpg * blk, hd), lambda h, qg: (h, qg, 0)),
            pl.BlockSpec((1, N, hd), lambda h, qg: (H + h, 0, 0)),
            pl.BlockSpec((1, N, hd), lambda h, qg: (2 * H + h, 0, 0)),
        ],
        out_specs=pl.BlockSpec((qpg * blk, hd), lambda h, qg: (qg, h)),
        out_shape=jax.ShapeDtypeStruct((N, C), jnp.bfloat16),
    )(idx, qkvs, qkvs, qkvs)

    bn = 512
    out = pl.pallas_call(
        _proj_kernel,
        grid=(C // bn,),
        in_specs=[
            pl.BlockSpec((N, C), lambda j: (0, 0)),
            pl.BlockSpec((C, bn), lambda j: (0, j)),
            pl.BlockSpec((1, bn), lambda j: (0, j)),
        ],
        out_specs=pl.BlockSpec((N, bn), lambda j: (0, j)),
        out_shape=jax.ShapeDtypeStruct((N, C), jnp.float32),
    )(attn, wpb, b_proj.reshape(1, C))
    return out.reshape(B, N, C)


# speedup vs baseline: 1.2492x; 1.2492x over previous
import functools
import math

import jax
import jax.numpy as jnp
from jax.experimental import pallas as pl
from jax.experimental.pallas import tpu as pltpu

_H = 16
_BLK = 64
_TOPK_FRAC = 0.5


def _dot(a, b, dims):
    return jax.lax.dot_general(a, b, dims, preferred_element_type=jnp.float32)


def _qkv_kernel(x_ref, w_ref, o_ref, m_ref, *, hd, heads_per_step, blk, nb):
    r = _dot(x_ref[...], w_ref[...], (((1,), (0,)), ((), ())))
    n = r.shape[0]
    m = jnp.sum(r.reshape(nb, blk, heads_per_step * hd), axis=1) * (1.0 / blk)
    for t in range(heads_per_step):
        o_ref[t] = r[:, t * hd:(t + 1) * hd].astype(jnp.bfloat16)
        m_ref[t] = m[:, t * hd:(t + 1) * hd]


def _select_kernel(qm_ref, km_ref, idx_ref, *, nb, kc):
    qb = qm_ref[0]
    kb = km_ref[0]
    sim = _dot(qb.astype(jnp.bfloat16), kb.astype(jnp.bfloat16),
               (((1,), (1,)), ((), ())))
    j_iota = jax.lax.broadcasted_iota(jnp.int32, (nb, nb), 1)
    count = jnp.zeros((nb, nb), jnp.float32)
    for jp in range(nb):
        coljp = sim[:, jp:jp + 1]
        beats = (coljp > sim) | ((coljp == sim) & (jp < j_iota))
        count = count + beats.astype(jnp.float32)
    bmask = (count < kc).astype(jnp.float32)
    lt = (jax.lax.broadcasted_iota(jnp.int32, (nb, nb), 0)
          < jax.lax.broadcasted_iota(jnp.int32, (nb, nb), 1)).astype(jnp.float32)
    pos = jnp.dot(bmask, lt, preferred_element_type=jnp.float32,
                  precision=jax.lax.Precision.HIGHEST)
    p_iota = jax.lax.broadcasted_iota(jnp.int32, (nb, kc, nb), 1).astype(jnp.float32)
    j3 = jax.lax.broadcasted_iota(jnp.int32, (nb, kc, nb), 2).astype(jnp.float32)
    oh = ((pos[:, None, :] == p_iota) & (bmask[:, None, :] > 0)).astype(jnp.float32)
    idxf = jnp.sum(j3 * oh, axis=2)
    idx_ref[0] = idxf.astype(jnp.int32)


def _attn_kernel(idx_ref, q_ref, k_ref, v_ref, o_ref, *, scale, blk, kc, qpg):
    h = pl.program_id(0)
    qg = pl.program_id(1)
    for t in range(qpg):
        qi = qg * qpg + t
        q = q_ref[0, pl.ds(t * blk, blk), :]
        ks = jnp.concatenate(
            [k_ref[0, pl.ds(idx_ref[h, qi, j] * blk, blk), :] for j in range(kc)],
            axis=0)
        vs = jnp.concatenate(
            [v_ref[0, pl.ds(idx_ref[h, qi, j] * blk, blk), :] for j in range(kc)],
            axis=0)
        s = _dot(q, ks, (((1,), (1,)), ((), ()))) * scale
        mx = jnp.max(s, axis=-1, keepdims=True)
        pexp = jnp.exp(s - mx)
        l = jnp.sum(pexp, axis=-1, keepdims=True)
        po = pexp.astype(jnp.bfloat16)
        o_ref[pl.ds(t * blk, blk), :] = (
            _dot(po, vs, (((1,), (0,)), ((), ()))) / l).astype(jnp.bfloat16)


def _proj_kernel(a_ref, w_ref, b_ref, o_ref):
    o_ref[...] = _dot(a_ref[...], w_ref[...], (((1,), (0,)), ((), ()))) + b_ref[...]


def kernel(x, W_qkv, W_proj, b_proj):
    B, N, C = x.shape
    H = _H
    hd = C // H
    blk = _BLK
    nb = N // blk
    kc = max(1, int(math.ceil(_TOPK_FRAC * nb)))
    scale = hd ** -0.5
    xb = x.reshape(N, C).astype(jnp.bfloat16)
    wqb = W_qkv.astype(jnp.bfloat16)
    wpb = W_proj.astype(jnp.bfloat16)

    hps = 4
    qkvs, means = pl.pallas_call(
        functools.partial(_qkv_kernel, hd=hd, heads_per_step=hps, blk=blk, nb=nb),
        grid=(3 * H // hps,),
        in_specs=[
            pl.BlockSpec((N, C), lambda j: (0, 0)),
            pl.BlockSpec((C, hps * hd), lambda j: (0, j)),
        ],
        out_specs=[
            pl.BlockSpec((hps, N, hd), lambda j: (j, 0, 0)),
            pl.BlockSpec((hps, nb, hd), lambda j: (j, 0, 0)),
        ],
        out_shape=[
            jax.ShapeDtypeStruct((3 * H, N, hd), jnp.bfloat16),
            jax.ShapeDtypeStruct((3 * H, nb, hd), jnp.float32),
        ],
    )(xb, wqb)

    idx = pl.pallas_call(
        functools.partial(_select_kernel, nb=nb, kc=kc),
        grid=(H,),
        in_specs=[
            pl.BlockSpec((1, nb, hd), lambda h: (h, 0, 0)),
            pl.BlockSpec((1, nb, hd), lambda h: (H + h, 0, 0)),
        ],
        out_specs=pl.BlockSpec((1, nb, kc), lambda h: (h, 0, 0)),
        out_shape=jax.ShapeDtypeStruct((H, nb, kc), jnp.int32),
    )(means, means)

    qpg = 8
    attn = pl.pallas_call(
        functools.partial(_attn_kernel, scale=scale, blk=blk, kc=kc, qpg=qpg),
        grid=(H, nb // qpg),
        in_specs=[
            pl.BlockSpec(memory_space=pltpu.SMEM),
            pl.BlockSpec((1, qpg * blk, hd), lambda h, qg: (h, qg, 0)),
            pl.BlockSpec((1, N, hd), lambda h, qg: (H + h, 0, 0)),
            pl.BlockSpec((1, N, hd), lambda h, qg: (2 * H + h, 0, 0)),
        ],
        out_specs=pl.BlockSpec((qpg * blk, hd), lambda h, qg: (qg, h)),
        out_shape=jax.ShapeDtypeStruct((N, C), jnp.bfloat16),
    )(idx, qkvs, qkvs, qkvs)

    bn = 512
    out = pl.pallas_call(
        _proj_kernel,
        grid=(C // bn,),
        in_specs=[
            pl.BlockSpec((N, C), lambda j: (0, 0)),
            pl.BlockSpec((C, bn), lambda j: (0, j)),
            pl.BlockSpec((1, bn), lambda j: (0, j)),
        ],
        out_specs=pl.BlockSpec((N, bn), lambda j: (0, j)),
        out_shape=jax.ShapeDtypeStruct((N, C), jnp.float32),
    )(attn, wpb, b_proj.reshape(1, C))
    return out.reshape(B, N, C)


# speedup vs baseline: 1.2617x; 1.0100x over previous
import functools
import math

import jax
import jax.numpy as jnp
from jax.experimental import pallas as pl
from jax.experimental.pallas import tpu as pltpu

_H = 16
_BLK = 64
_TOPK_FRAC = 0.5


def _dot(a, b, dims):
    return jax.lax.dot_general(a, b, dims, preferred_element_type=jnp.float32)


def _qkv_kernel(x_ref, w_ref, o_ref, m_ref, *, hd, heads_per_step, blk, nb):
    r = _dot(x_ref[...], w_ref[...], (((1,), (0,)), ((), ())))
    n = r.shape[0]
    m = jnp.sum(r.reshape(nb, blk, heads_per_step * hd), axis=1) * (1.0 / blk)
    for t in range(heads_per_step):
        o_ref[t] = r[:, t * hd:(t + 1) * hd].astype(jnp.bfloat16)
        m_ref[t] = m[:, t * hd:(t + 1) * hd]


def _select_kernel(qm_ref, km_ref, idx_ref, *, nb, kc):
    qb = qm_ref[0]
    kb = km_ref[0]
    sim = _dot(qb.astype(jnp.bfloat16), kb.astype(jnp.bfloat16),
               (((1,), (1,)), ((), ())))
    j_iota = jax.lax.broadcasted_iota(jnp.int32, (nb, nb), 1)
    count = jnp.zeros((nb, nb), jnp.float32)
    for jp in range(nb):
        coljp = sim[:, jp:jp + 1]
        beats = (coljp > sim) | ((coljp == sim) & (jp < j_iota))
        count = count + beats.astype(jnp.float32)
    bmask = (count < kc).astype(jnp.float32)
    lt = (jax.lax.broadcasted_iota(jnp.int32, (nb, nb), 0)
          < jax.lax.broadcasted_iota(jnp.int32, (nb, nb), 1)).astype(jnp.float32)
    pos = jnp.dot(bmask, lt, preferred_element_type=jnp.float32,
                  precision=jax.lax.Precision.HIGHEST)
    p_iota = jax.lax.broadcasted_iota(jnp.int32, (nb, kc, nb), 1).astype(jnp.float32)
    j3 = jax.lax.broadcasted_iota(jnp.int32, (nb, kc, nb), 2).astype(jnp.float32)
    oh = ((pos[:, None, :] == p_iota) & (bmask[:, None, :] > 0)).astype(jnp.float32)
    idxf = jnp.sum(j3 * oh, axis=2)
    idx_ref[0] = idxf.astype(jnp.int32)


def _attn_kernel(idx_ref, q_ref, k_ref, v_ref, o_ref, *, scale, blk, kc, qpg):
    h = pl.program_id(0)
    qg = pl.program_id(1)
    for t in range(qpg):
        qi = qg * qpg + t
        q = q_ref[0, pl.ds(t * blk, blk), :]
        ks = jnp.concatenate(
            [k_ref[0, pl.ds(idx_ref[h, qi, j] * blk, blk), :] for j in range(kc)],
            axis=0)
        vs = jnp.concatenate(
            [v_ref[0, pl.ds(idx_ref[h, qi, j] * blk, blk), :] for j in range(kc)],
            axis=0)
        s = _dot(q, ks, (((1,), (1,)), ((), ()))) * scale
        mx = jnp.max(s, axis=-1, keepdims=True)
        pexp = jnp.exp(s - mx)
        l = jnp.sum(pexp, axis=-1, keepdims=True)
        po = pexp.astype(jnp.bfloat16)
        o_ref[pl.ds(t * blk, blk), :] = (
            _dot(po, vs, (((1,), (0,)), ((), ()))) / l).astype(jnp.bfloat16)


def _proj_kernel(a_ref, w_ref, b_ref, o_ref):
    o_ref[...] = _dot(a_ref[...], w_ref[...], (((1,), (0,)), ((), ()))) + b_ref[...]


def kernel(x, W_qkv, W_proj, b_proj):
    B, N, C = x.shape
    H = _H
    hd = C // H
    blk = _BLK
    nb = N // blk
    kc = max(1, int(math.ceil(_TOPK_FRAC * nb)))
    scale = hd ** -0.5
    xb = x.reshape(N, C).astype(jnp.bfloat16)
    wqb = W_qkv.astype(jnp.bfloat16)
    wpb = W_proj.astype(jnp.bfloat16)

    hps = 4
    qkvs, means = pl.pallas_call(
        functools.partial(_qkv_kernel, hd=hd, heads_per_step=hps, blk=blk, nb=nb),
        grid=(3 * H // hps,),
        in_specs=[
            pl.BlockSpec((N, C), lambda j: (0, 0)),
            pl.BlockSpec((C, hps * hd), lambda j: (0, j)),
        ],
        out_specs=[
            pl.BlockSpec((hps, N, hd), lambda j: (j, 0, 0)),
            pl.BlockSpec((hps, nb, hd), lambda j: (j, 0, 0)),
        ],
        out_shape=[
            jax.ShapeDtypeStruct((3 * H, N, hd), jnp.bfloat16),
            jax.ShapeDtypeStruct((3 * H, nb, hd), jnp.float32),
        ],
    )(xb, wqb)

    idx = pl.pallas_call(
        functools.partial(_select_kernel, nb=nb, kc=kc),
        grid=(H,),
        in_specs=[
            pl.BlockSpec((1, nb, hd), lambda h: (h, 0, 0)),
            pl.BlockSpec((1, nb, hd), lambda h: (H + h, 0, 0)),
        ],
        out_specs=pl.BlockSpec((1, nb, kc), lambda h: (h, 0, 0)),
        out_shape=jax.ShapeDtypeStruct((H, nb, kc), jnp.int32),
    )(means, means)

    qpg = 16
    attn = pl.pallas_call(
        functools.partial(_attn_kernel, scale=scale, blk=blk, kc=kc, qpg=qpg),
        grid=(H, nb // qpg),
        in_specs=[
            pl.BlockSpec(memory_space=pltpu.SMEM),
            pl.BlockSpec((1, qpg * blk, hd), lambda h, qg: (h, qg, 0)),
            pl.BlockSpec((1, N, hd), lambda h, qg: (H + h, 0, 0)),
            pl.BlockSpec((1, N, hd), lambda h, qg: (2 * H + h, 0, 0)),
        ],
        out_specs=pl.BlockSpec((qpg * blk, hd), lambda h, qg: (qg, h)),
        out_shape=jax.ShapeDtypeStruct((N, C), jnp.bfloat16),
    )(idx, qkvs, qkvs, qkvs)

    bn = 512
    out = pl.pallas_call(
        _proj_kernel,
        grid=(C // bn,),
        in_specs=[
            pl.BlockSpec((N, C), lambda j: (0, 0)),
            pl.BlockSpec((C, bn), lambda j: (0, j)),
            pl.BlockSpec((1, bn), lambda j: (0, j)),
        ],
        out_specs=pl.BlockSpec((N, bn), lambda j: (0, j)),
        out_shape=jax.ShapeDtypeStruct((N, C), jnp.float32),
    )(attn, wpb, b_proj.reshape(1, C))
    return out.reshape(B, N, C)


# speedup vs baseline: 1.2786x; 1.0135x over previous
import functools
import math

import jax
import jax.numpy as jnp
from jax.experimental import pallas as pl
from jax.experimental.pallas import tpu as pltpu

_H = 16
_BLK = 64
_TOPK_FRAC = 0.5


def _dot(a, b, dims):
    return jax.lax.dot_general(a, b, dims, preferred_element_type=jnp.float32)


def _qkv_kernel(x_ref, w_ref, o_ref, m_ref, *, hd, heads_per_step, blk, nb):
    r = _dot(x_ref[...], w_ref[...], (((1,), (0,)), ((), ())))
    n = r.shape[0]
    m = jnp.sum(r.reshape(nb, blk, heads_per_step * hd), axis=1) * (1.0 / blk)
    for t in range(heads_per_step):
        o_ref[t] = r[:, t * hd:(t + 1) * hd].astype(jnp.bfloat16)
        m_ref[t] = m[:, t * hd:(t + 1) * hd]


def _select_kernel(qm_ref, km_ref, idx_ref, *, nb, kc):
    qb = qm_ref[0]
    kb = km_ref[0]
    sim = _dot(qb.astype(jnp.bfloat16), kb.astype(jnp.bfloat16),
               (((1,), (1,)), ((), ())))
    j_iota = jax.lax.broadcasted_iota(jnp.int32, (nb, nb), 1)
    count = jnp.zeros((nb, nb), jnp.float32)
    for jp in range(nb):
        coljp = sim[:, jp:jp + 1]
        beats = (coljp > sim) | ((coljp == sim) & (jp < j_iota))
        count = count + beats.astype(jnp.float32)
    bmask = (count < kc).astype(jnp.float32)
    lt = (jax.lax.broadcasted_iota(jnp.int32, (nb, nb), 0)
          < jax.lax.broadcasted_iota(jnp.int32, (nb, nb), 1)).astype(jnp.float32)
    pos = jnp.dot(bmask, lt, preferred_element_type=jnp.float32,
                  precision=jax.lax.Precision.HIGHEST)
    p_iota = jax.lax.broadcasted_iota(jnp.int32, (nb, kc, nb), 1).astype(jnp.float32)
    j3 = jax.lax.broadcasted_iota(jnp.int32, (nb, kc, nb), 2).astype(jnp.float32)
    oh = ((pos[:, None, :] == p_iota) & (bmask[:, None, :] > 0)).astype(jnp.float32)
    idxf = jnp.sum(j3 * oh, axis=2)
    idx_ref[0] = idxf.astype(jnp.int32)


def _attn_kernel(idx_ref, q_ref, k_ref, v_ref, o_ref, *, scale, blk, kc, qpg):
    h = pl.program_id(0)
    qg = pl.program_id(1)
    for t in range(qpg):
        qi = qg * qpg + t
        q = q_ref[0, pl.ds(t * blk, blk), :]
        ks = jnp.concatenate(
            [k_ref[0, pl.ds(idx_ref[h, qi, j] * blk, blk), :] for j in range(kc)],
            axis=0)
        vs = jnp.concatenate(
            [v_ref[0, pl.ds(idx_ref[h, qi, j] * blk, blk), :] for j in range(kc)],
            axis=0)
        s = _dot(q, ks, (((1,), (1,)), ((), ()))) * scale
        mx = jnp.max(s, axis=-1, keepdims=True)
        pexp = jnp.exp(s - mx)
        l = jnp.sum(pexp, axis=-1, keepdims=True)
        po = pexp.astype(jnp.bfloat16)
        o_ref[pl.ds(t * blk, blk), :] = (
            _dot(po, vs, (((1,), (0,)), ((), ()))) / l).astype(jnp.bfloat16)


def _proj_kernel(a_ref, w_ref, b_ref, o_ref):
    o_ref[...] = _dot(a_ref[...], w_ref[...], (((1,), (0,)), ((), ()))) + b_ref[...]


def kernel(x, W_qkv, W_proj, b_proj):
    B, N, C = x.shape
    H = _H
    hd = C // H
    blk = _BLK
    nb = N // blk
    kc = max(1, int(math.ceil(_TOPK_FRAC * nb)))
    scale = hd ** -0.5
    xb = x.reshape(N, C).astype(jnp.bfloat16)
    wqb = W_qkv.astype(jnp.bfloat16)
    wpb = W_proj.astype(jnp.bfloat16)

    hps = 4
    qkvs, means = pl.pallas_call(
        functools.partial(_qkv_kernel, hd=hd, heads_per_step=hps, blk=blk, nb=nb),
        grid=(3 * H // hps,),
        in_specs=[
            pl.BlockSpec((N, C), lambda j: (0, 0)),
            pl.BlockSpec((C, hps * hd), lambda j: (0, j)),
        ],
        out_specs=[
            pl.BlockSpec((hps, N, hd), lambda j: (j, 0, 0)),
            pl.BlockSpec((hps, nb, hd), lambda j: (j, 0, 0)),
        ],
        out_shape=[
            jax.ShapeDtypeStruct((3 * H, N, hd), jnp.bfloat16),
            jax.ShapeDtypeStruct((3 * H, nb, hd), jnp.float32),
        ],
    )(xb, wqb)

    idx = pl.pallas_call(
        functools.partial(_select_kernel, nb=nb, kc=kc),
        grid=(H,),
        in_specs=[
            pl.BlockSpec((1, nb, hd), lambda h: (h, 0, 0)),
            pl.BlockSpec((1, nb, hd), lambda h: (H + h, 0, 0)),
        ],
        out_specs=pl.BlockSpec((1, nb, kc), lambda h: (h, 0, 0)),
        out_shape=jax.ShapeDtypeStruct((H, nb, kc), jnp.int32),
    )(means, means)

    qpg = 32
    attn = pl.pallas_call(
        functools.partial(_attn_kernel, scale=scale, blk=blk, kc=kc, qpg=qpg),
        grid=(H, nb // qpg),
        in_specs=[
            pl.BlockSpec(memory_space=pltpu.SMEM),
            pl.BlockSpec((1, qpg * blk, hd), lambda h, qg: (h, qg, 0)),
            pl.BlockSpec((1, N, hd), lambda h, qg: (H + h, 0, 0)),
            pl.BlockSpec((1, N, hd), lambda h, qg: (2 * H + h, 0, 0)),
        ],
        out_specs=pl.BlockSpec((qpg * blk, hd), lambda h, qg: (qg, h)),
        out_shape=jax.ShapeDtypeStruct((N, C), jnp.bfloat16),
    )(idx, qkvs, qkvs, qkvs)

    bn = 512
    out = pl.pallas_call(
        _proj_kernel,
        grid=(C // bn,),
        in_specs=[
            pl.BlockSpec((N, C), lambda j: (0, 0)),
            pl.BlockSpec((C, bn), lambda j: (0, j)),
            pl.BlockSpec((1, bn), lambda j: (0, j)),
        ],
        out_specs=pl.BlockSpec((N, bn), lambda j: (0, j)),
        out_shape=jax.ShapeDtypeStruct((N, C), jnp.float32),
    )(attn, wpb, b_proj.reshape(1, C))
    return out.reshape(B, N, C)


# speedup vs baseline: 1.4414x; 1.1273x over previous
import functools
import math

import jax
import jax.numpy as jnp
from jax.experimental import pallas as pl
from jax.experimental.pallas import tpu as pltpu

_H = 16
_BLK = 64
_TOPK_FRAC = 0.5


def _dot(a, b, dims):
    return jax.lax.dot_general(a, b, dims, preferred_element_type=jnp.float32)


def _qkv_kernel(x_ref, w_ref, o_ref, m_ref, xb_scr, *, hd, heads_per_step, blk, nb):
    @pl.when(pl.program_id(0) == 0)
    def _():
        xb_scr[...] = x_ref[...].astype(jnp.bfloat16)
    r = _dot(xb_scr[...], w_ref[...].astype(jnp.bfloat16),
             (((1,), (0,)), ((), ())))
    m = jnp.sum(r.reshape(nb, blk, heads_per_step * hd), axis=1) * (1.0 / blk)
    for t in range(heads_per_step):
        o_ref[t] = r[:, t * hd:(t + 1) * hd].astype(jnp.bfloat16)
        m_ref[t] = m[:, t * hd:(t + 1) * hd]


def _select_kernel(qm_ref, km_ref, idx_ref, *, nb, kc):
    qb = qm_ref[0]
    kb = km_ref[0]
    sim = _dot(qb.astype(jnp.bfloat16), kb.astype(jnp.bfloat16),
               (((1,), (1,)), ((), ())))
    j_iota = jax.lax.broadcasted_iota(jnp.int32, (nb, nb), 1)
    count = jnp.zeros((nb, nb), jnp.float32)
    for jp in range(nb):
        coljp = sim[:, jp:jp + 1]
        beats = (coljp > sim) | ((coljp == sim) & (jp < j_iota))
        count = count + beats.astype(jnp.float32)
    bmask = (count < kc).astype(jnp.float32)
    lt = (jax.lax.broadcasted_iota(jnp.int32, (nb, nb), 0)
          < jax.lax.broadcasted_iota(jnp.int32, (nb, nb), 1)).astype(jnp.float32)
    pos = jnp.dot(bmask, lt, preferred_element_type=jnp.float32,
                  precision=jax.lax.Precision.HIGHEST)
    p_iota = jax.lax.broadcasted_iota(jnp.int32, (nb, kc, nb), 1).astype(jnp.float32)
    j3 = jax.lax.broadcasted_iota(jnp.int32, (nb, kc, nb), 2).astype(jnp.float32)
    oh = ((pos[:, None, :] == p_iota) & (bmask[:, None, :] > 0)).astype(jnp.float32)
    idxf = jnp.sum(j3 * oh, axis=2)
    idx_ref[0] = idxf.astype(jnp.int32)


def _attn_kernel(idx_ref, q_ref, k_ref, v_ref, o_ref, *, scale, blk, kc, qpg):
    h = pl.program_id(0)
    qg = pl.program_id(1)
    for t in range(qpg):
        qi = qg * qpg + t
        q = q_ref[0, pl.ds(t * blk, blk), :]
        ks = jnp.concatenate(
            [k_ref[0, pl.ds(idx_ref[h, qi, j] * blk, blk), :] for j in range(kc)],
            axis=0)
        vs = jnp.concatenate(
            [v_ref[0, pl.ds(idx_ref[h, qi, j] * blk, blk), :] for j in range(kc)],
            axis=0)
        s = _dot(q, ks, (((1,), (1,)), ((), ()))) * scale
        mx = jnp.max(s, axis=-1, keepdims=True)
        pexp = jnp.exp(s - mx)
        l = jnp.sum(pexp, axis=-1, keepdims=True)
        po = pexp.astype(jnp.bfloat16)
        o_ref[pl.ds(t * blk, blk), :] = (
            _dot(po, vs, (((1,), (0,)), ((), ()))) / l).astype(jnp.bfloat16)


def _proj_kernel(a_ref, w_ref, b_ref, o_ref):
    o_ref[...] = _dot(a_ref[...], w_ref[...].astype(jnp.bfloat16),
                      (((1,), (0,)), ((), ()))) + b_ref[...]


def kernel(x, W_qkv, W_proj, b_proj):
    B, N, C = x.shape
    H = _H
    hd = C // H
    blk = _BLK
    nb = N // blk
    kc = max(1, int(math.ceil(_TOPK_FRAC * nb)))
    scale = hd ** -0.5
    x2 = x.reshape(N, C)

    hps = 4
    qkvs, means = pl.pallas_call(
        functools.partial(_qkv_kernel, hd=hd, heads_per_step=hps, blk=blk, nb=nb),
        grid=(3 * H // hps,),
        in_specs=[
            pl.BlockSpec((N, C), lambda j: (0, 0)),
            pl.BlockSpec((C, hps * hd), lambda j: (0, j)),
        ],
        out_specs=[
            pl.BlockSpec((hps, N, hd), lambda j: (j, 0, 0)),
            pl.BlockSpec((hps, nb, hd), lambda j: (j, 0, 0)),
        ],
        out_shape=[
            jax.ShapeDtypeStruct((3 * H, N, hd), jnp.bfloat16),
            jax.ShapeDtypeStruct((3 * H, nb, hd), jnp.float32),
        ],
        scratch_shapes=[pltpu.VMEM((N, C), jnp.bfloat16)],
    )(x2, W_qkv)

    idx = pl.pallas_call(
        functools.partial(_select_kernel, nb=nb, kc=kc),
        grid=(H,),
        in_specs=[
            pl.BlockSpec((1, nb, hd), lambda h: (h, 0, 0)),
            pl.BlockSpec((1, nb, hd), lambda h: (H + h, 0, 0)),
        ],
        out_specs=pl.BlockSpec((1, nb, kc), lambda h: (h, 0, 0)),
        out_shape=jax.ShapeDtypeStruct((H, nb, kc), jnp.int32),
    )(means, means)

    qpg = 32
    attn = pl.pallas_call(
        functools.partial(_attn_kernel, scale=scale, blk=blk, kc=kc, qpg=qpg),
        grid=(H, nb // qpg),
        in_specs=[
            pl.BlockSpec(memory_space=pltpu.SMEM),
            pl.BlockSpec((1, qpg * blk, hd), lambda h, qg: (h, qg, 0)),
            pl.BlockSpec((1, N, hd), lambda h, qg: (H + h, 0, 0)),
            pl.BlockSpec((1, N, hd), lambda h, qg: (2 * H + h, 0, 0)),
        ],
        out_specs=pl.BlockSpec((qpg * blk, hd), lambda h, qg: (qg, h)),
        out_shape=jax.ShapeDtypeStruct((N, C), jnp.bfloat16),
    )(idx, qkvs, qkvs, qkvs)

    bn = 512
    out = pl.pallas_call(
        _proj_kernel,
        grid=(C // bn,),
        in_specs=[
            pl.BlockSpec((N, C), lambda j: (0, 0)),
            pl.BlockSpec((C, bn), lambda j: (0, j)),
            pl.BlockSpec((1, bn), lambda j: (0, j)),
        ],
        out_specs=pl.BlockSpec((N, bn), lambda j: (0, j)),
        out_shape=jax.ShapeDtypeStruct((N, C), jnp.float32),
    )(attn, W_proj, b_proj.reshape(1, C))
    return out.reshape(B, N, C)
